# baseline (speedup 1.0000x reference)
import jax
import jax.numpy as jnp
from jax import lax
from jax.experimental import pallas as pl
from jax.experimental.pallas import tpu as pltpu
from jax.experimental.pallas import tpu_sc as plsc

_PRESENCE = 1.0
_FREQUENCY = 0.5
_DECAY = 0.1
_V = 100000
_SEQ = 128
_T = 16384
_NS = 16
_C = 6272
_LASTBASE = _V - _C
_TPT = _T // _NS
_ROWS = _TPT // 128
_NEG = -3.0e38
_IMAX = 2**31 - 1


def _body(rows_hbm, tok_hbm, cnt_hbm, out_row, out_cnt, out_tok,
          stg_m, stg_i, acc, lrow, tok_v, cnt_v, pen_v,
          mbuf, ibuf, lm, li, ncnt_v, sem_row, sem_io, sem_s):
    wid = lax.axis_index("s")
    base = pl.multiple_of(jnp.minimum(wid * _C, _LASTBASE), 32)
    trow = wid * _ROWS
    lanes = lax.broadcasted_iota(jnp.int32, (16,), 0)

    cp_row = pltpu.async_copy(rows_hbm.at[pl.ds((_SEQ - 1) * _V + base, _C)],
                              lrow, sem_row)
    cp_tok = pltpu.async_copy(tok_hbm.at[pl.ds(trow, _ROWS)], tok_v, sem_io)
    cp_cnt = pltpu.async_copy(cnt_hbm.at[pl.ds(trow, _ROWS)], cnt_v, sem_io)
    cp_tok.wait()
    cp_cnt.wait()

    for j in range(_ROWS):
        def pbody(k, _, j=j):
            c = cnt_v[j, pl.ds(k * 16, 16)]
            pen_v[j, pl.ds(k * 16, 16)] = -_PRESENCE - _FREQUENCY * c
            return 0
        lax.fori_loop(0, 128 // 16, pbody, 0)

    cp_row.wait()
    pltpu.sync_copy(lrow, acc.at[pl.ds(base, _C)])
    plsc.subcore_barrier()

    scats = [pltpu.async_copy(pen_v.at[j], acc.at[tok_v.at[j]], sem_s,
                              add=True) for j in range(_ROWS)]
    for cp in scats:
        cp.wait()

    plsc.subcore_barrier()

    pltpu.sync_copy(acc.at[pl.ds(base, _C)], lrow)

    def am_body(i, carry):
        m, mi = carry
        v = lrow[pl.ds(i * 16, 16)]
        upd = v > m
        return jnp.where(upd, v, m), jnp.where(upd, i, mi)

    m0 = jnp.full((16,), _NEG, jnp.float32)
    i0 = jnp.zeros((16,), jnp.int32)
    m, mi = lax.fori_loop(0, _C // 16, am_body, (m0, i0))

    pos = mi * 16 + lanes + base
    bm = jnp.max(m)
    bp = jnp.min(jnp.where(m == bm, pos, _IMAX))

    cp_out = pltpu.async_copy(lrow, out_row.at[pl.ds(base, _C)], sem_row)

    mbuf[...] = jnp.where(lanes == wid, bm, _NEG)
    ibuf[...] = jnp.where(lanes == wid, bp, _IMAX)
    pltpu.sync_copy(mbuf, stg_m.at[wid])
    pltpu.sync_copy(ibuf, stg_i.at[wid])
    plsc.subcore_barrier()
    pltpu.sync_copy(stg_m, lm)
    pltpu.sync_copy(stg_i, li)

    macc = lm[0]
    pacc = li[0]
    for j in range(1, _NS):
        macc = jnp.maximum(macc, lm[j])
        pacc = jnp.minimum(pacc, li[j])
    gbest = jnp.max(macc)
    gpos = jnp.min(jnp.where(macc == gbest, pacc, _IMAX))

    for j in range(_ROWS):
        def cbody(k, _, j=j):
            t = tok_v[j, pl.ds(k * 16, 16)]
            c = cnt_v[j, pl.ds(k * 16, 16)]
            nc = c * (1.0 - _DECAY) + jnp.where(t == gpos, 1.0, 0.0)
            ncnt_v[j, pl.ds(k * 16, 16)] = nc
            return 0
        lax.fori_loop(0, 128 // 16, cbody, 0)
    pltpu.sync_copy(ncnt_v, out_cnt.at[pl.ds(trow, _ROWS)])

    @pl.when(wid == 0)
    def _():
        ibuf[...] = jnp.full((16,), gpos, jnp.int32)
        pltpu.sync_copy(ibuf.at[pl.ds(0, 1)], out_tok)

    cp_out.wait()


@jax.jit
def _run(rows, tok2d, cnt2d):
    mesh = plsc.VectorSubcoreMesh(
        core_axis_name="c", subcore_axis_name="s", num_cores=1)
    f = pl.kernel(
        _body,
        out_type=(
            jax.ShapeDtypeStruct((_V,), jnp.float32),
            jax.ShapeDtypeStruct((_T // 128, 128), jnp.float32),
            jax.ShapeDtypeStruct((1,), jnp.int32),
            jax.ShapeDtypeStruct((_NS, 16), jnp.float32),
            jax.ShapeDtypeStruct((_NS, 16), jnp.int32),
        ),
        mesh=mesh,
        compiler_params=pltpu.CompilerParams(needs_layout_passes=False),
        scratch_types=[
            pltpu.VMEM_SHARED((_V,), jnp.float32),
            pltpu.VMEM((_C,), jnp.float32),
            pltpu.VMEM((_ROWS, 128), jnp.int32),
            pltpu.VMEM((_ROWS, 128), jnp.float32),
            pltpu.VMEM((_ROWS, 128), jnp.float32),
            pltpu.VMEM((16,), jnp.float32),
            pltpu.VMEM((16,), jnp.int32),
            pltpu.VMEM((_NS, 16), jnp.float32),
            pltpu.VMEM((_NS, 16), jnp.int32),
            pltpu.VMEM((_ROWS, 128), jnp.float32),
            pltpu.SemaphoreType.DMA,
            pltpu.SemaphoreType.DMA,
            pltpu.SemaphoreType.DMA,
        ],
    )
    return f(rows, tok2d, cnt2d)


def kernel(logits, prev_tokens, counts):
    rows = logits.reshape(_SEQ * _V)
    tok2d = prev_tokens.reshape(_T // 128, 128)
    cnt2d = counts.reshape(_T // 128, 128)
    out_row, out_cnt, out_tok, _, _ = _run(rows, tok2d, cnt2d)
    return (out_tok, out_row, out_cnt.reshape(_T))


# speedup vs baseline: 2.5429x; 2.5429x over previous
import jax
import jax.numpy as jnp
from jax import lax
from jax.experimental import pallas as pl
from jax.experimental.pallas import tpu as pltpu
from jax.experimental.pallas import tpu_sc as plsc

_PRESENCE = 1.0
_FREQUENCY = 0.5
_DECAY = 0.1
_V = 100000
_SEQ = 128
_T = 16384
_NS = 16
_C = 6272
_LASTBASE = _V - _C
_TPT = _T // _NS
_ROWS = _TPT // 128
_NEG = -3.0e38
_IMAX = 2**31 - 1


def _body(rows_hbm, tok_hbm, cnt_hbm, out_row, out_cnt, out_tok,
          stg_m, stg_i, acc, lrow, tok_v, cnt_v, pen_v,
          mbuf, ibuf, lm, li, ncnt_v, sem_row, sem_io, sem_s):
    wid = lax.axis_index("s")
    base = pl.multiple_of(jnp.minimum(wid * _C, _LASTBASE), 32)
    trow = wid * _ROWS
    lanes = lax.broadcasted_iota(jnp.int32, (16,), 0)

    cp_row = pltpu.async_copy(rows_hbm.at[pl.ds(base, _C)], lrow, sem_row)
    cp_tok = pltpu.async_copy(tok_hbm.at[pl.ds(trow, _ROWS)], tok_v, sem_io)
    cp_cnt = pltpu.async_copy(cnt_hbm.at[pl.ds(trow, _ROWS)], cnt_v, sem_io)
    cp_tok.wait()
    cp_cnt.wait()

    for j in range(_ROWS):
        def pbody(k, _, j=j):
            c = cnt_v[j, pl.ds(k * 16, 16)]
            pen_v[j, pl.ds(k * 16, 16)] = -_PRESENCE - _FREQUENCY * c
            return 0
        lax.fori_loop(0, 128 // 16, pbody, 0)

    cp_row.wait()
    pltpu.sync_copy(lrow, acc.at[pl.ds(base, _C)])
    plsc.subcore_barrier()

    scats = [pltpu.async_copy(pen_v.at[j], acc.at[tok_v.at[j]], sem_s,
                              add=True) for j in range(_ROWS)]
    for cp in scats:
        cp.wait()

    plsc.subcore_barrier()

    pltpu.sync_copy(acc.at[pl.ds(base, _C)], lrow)

    def am_body(i, carry):
        m, mi = carry
        v = lrow[pl.ds(i * 16, 16)]
        upd = v > m
        return jnp.where(upd, v, m), jnp.where(upd, i, mi)

    m0 = jnp.full((16,), _NEG, jnp.float32)
    i0 = jnp.zeros((16,), jnp.int32)
    m, mi = lax.fori_loop(0, _C // 16, am_body, (m0, i0))

    pos = mi * 16 + lanes + base
    bm = jnp.max(m)
    bp = jnp.min(jnp.where(m == bm, pos, _IMAX))

    cp_out = pltpu.async_copy(lrow, out_row.at[pl.ds(base, _C)], sem_row)

    mbuf[...] = jnp.where(lanes == wid, bm, _NEG)
    ibuf[...] = jnp.where(lanes == wid, bp, _IMAX)
    pltpu.sync_copy(mbuf, stg_m.at[wid])
    pltpu.sync_copy(ibuf, stg_i.at[wid])
    plsc.subcore_barrier()
    pltpu.sync_copy(stg_m, lm)
    pltpu.sync_copy(stg_i, li)

    macc = lm[0]
    pacc = li[0]
    for j in range(1, _NS):
        macc = jnp.maximum(macc, lm[j])
        pacc = jnp.minimum(pacc, li[j])
    gbest = jnp.max(macc)
    gpos = jnp.min(jnp.where(macc == gbest, pacc, _IMAX))

    for j in range(_ROWS):
        def cbody(k, _, j=j):
            t = tok_v[j, pl.ds(k * 16, 16)]
            c = cnt_v[j, pl.ds(k * 16, 16)]
            nc = c * (1.0 - _DECAY) + jnp.where(t == gpos, 1.0, 0.0)
            ncnt_v[j, pl.ds(k * 16, 16)] = nc
            return 0
        lax.fori_loop(0, 128 // 16, cbody, 0)
    pltpu.sync_copy(ncnt_v, out_cnt.at[pl.ds(trow, _ROWS)])

    @pl.when(wid == 0)
    def _():
        ibuf[...] = jnp.full((16,), gpos, jnp.int32)
        pltpu.sync_copy(ibuf.at[pl.ds(0, 1)], out_tok)

    cp_out.wait()


@jax.jit
def _run(rows, tok2d, cnt2d):
    mesh = plsc.VectorSubcoreMesh(
        core_axis_name="c", subcore_axis_name="s", num_cores=1)
    f = pl.kernel(
        _body,
        out_type=(
            jax.ShapeDtypeStruct((_V,), jnp.float32),
            jax.ShapeDtypeStruct((_T // 128, 128), jnp.float32),
            jax.ShapeDtypeStruct((1,), jnp.int32),
            jax.ShapeDtypeStruct((_NS, 16), jnp.float32),
            jax.ShapeDtypeStruct((_NS, 16), jnp.int32),
        ),
        mesh=mesh,
        compiler_params=pltpu.CompilerParams(needs_layout_passes=False),
        scratch_types=[
            pltpu.VMEM_SHARED((_V,), jnp.float32),
            pltpu.VMEM((_C,), jnp.float32),
            pltpu.VMEM((_ROWS, 128), jnp.int32),
            pltpu.VMEM((_ROWS, 128), jnp.float32),
            pltpu.VMEM((_ROWS, 128), jnp.float32),
            pltpu.VMEM((16,), jnp.float32),
            pltpu.VMEM((16,), jnp.int32),
            pltpu.VMEM((_NS, 16), jnp.float32),
            pltpu.VMEM((_NS, 16), jnp.int32),
            pltpu.VMEM((_ROWS, 128), jnp.float32),
            pltpu.SemaphoreType.DMA,
            pltpu.SemaphoreType.DMA,
            pltpu.SemaphoreType.DMA,
        ],
    )
    return f(rows, tok2d, cnt2d)


def kernel(logits, prev_tokens, counts):
    rows = logits[0, -1, :]
    tok2d = prev_tokens.reshape(_T // 128, 128)
    cnt2d = counts.reshape(_T // 128, 128)
    out_row, out_cnt, out_tok, _, _ = _run(rows, tok2d, cnt2d)
    return (out_tok, out_row, out_cnt.reshape(_T))
